# baseline (speedup 1.0000x reference)
import functools

import jax
import jax.numpy as jnp
from jax import lax
from jax.experimental import pallas as pl
from jax.experimental.pallas import tpu as pltpu
from jax.experimental.pallas import tpu_sc as plsc

BATCH = 16384
EMBED = 64
HIDDEN = 256

NC, NS = 2, 16
NW = NC * NS
BPW = BATCH // NW
ICHUNK = 128
NCHUNK = BPW // ICHUNK

_SC_MESH = plsc.VectorSubcoreMesh(
    core_axis_name="c", subcore_axis_name="s", num_cores=NC, num_subcores=NS
)


@functools.partial(
    pl.kernel,
    out_type=(
        jax.ShapeDtypeStruct((BATCH, EMBED), jnp.float32),
        jax.ShapeDtypeStruct((BATCH, EMBED), jnp.float32),
    ),
    mesh=_SC_MESH,
    scratch_types=[
        pltpu.VMEM((NCHUNK, ICHUNK), jnp.int32),
        pltpu.VMEM((NCHUNK, ICHUNK), jnp.int32),
        pltpu.VMEM((BPW, EMBED), jnp.float32),
        pltpu.VMEM((BPW, EMBED), jnp.float32),
        pltpu.SemaphoreType.DMA,
        pltpu.SemaphoreType.DMA,
    ],
    compiler_params=pltpu.CompilerParams(use_tc_tiling_on_sc=False),
)
def _sc_gather(u_tab, i_tab, u_idx, i_idx, u_out, i_out,
               uidx_v, iidx_v, urows_v, irows_v, usem, isem):
    wid = lax.axis_index("s") * NC + lax.axis_index("c")
    base = wid * BPW
    pltpu.sync_copy(u_idx.at[pl.ds(wid * NCHUNK, NCHUNK)], uidx_v)
    pltpu.sync_copy(i_idx.at[pl.ds(wid * NCHUNK, NCHUNK)], iidx_v)
    copies = []
    for j in range(NCHUNK):
        copies.append(pltpu.async_copy(
            u_tab.at[uidx_v.at[j]], urows_v.at[pl.ds(j * ICHUNK, ICHUNK)], usem))
        copies.append(pltpu.async_copy(
            i_tab.at[iidx_v.at[j]], irows_v.at[pl.ds(j * ICHUNK, ICHUNK)], isem))
    for c in copies:
        c.wait()
    pltpu.sync_copy(urows_v, u_out.at[pl.ds(base, BPW)])
    pltpu.sync_copy(irows_v, i_out.at[pl.ds(base, BPW)])


BLK = 1024


def _mlp_body(u_ref, i_ref, w1u_ref, w1i_ref, b1_ref, w2_ref, b2_ref, o_ref):
    h = (jnp.dot(u_ref[...], w1u_ref[...], preferred_element_type=jnp.float32)
         + jnp.dot(i_ref[...], w1i_ref[...], preferred_element_type=jnp.float32)
         + b1_ref[...])
    h = jnp.maximum(h, 0.0)
    o_ref[...] = (jnp.dot(h, w2_ref[...], preferred_element_type=jnp.float32)
                  + b2_ref[...])


@jax.jit
def _tc_mlp(u_emb, i_emb, w1u, w1i, b1, w2, b2):
    return pl.pallas_call(
        _mlp_body,
        grid=(BATCH // BLK,),
        in_specs=[
            pl.BlockSpec((BLK, EMBED), lambda i: (i, 0)),
            pl.BlockSpec((BLK, EMBED), lambda i: (i, 0)),
            pl.BlockSpec((EMBED, HIDDEN), lambda i: (0, 0)),
            pl.BlockSpec((EMBED, HIDDEN), lambda i: (0, 0)),
            pl.BlockSpec((1, HIDDEN), lambda i: (0, 0)),
            pl.BlockSpec((HIDDEN, 1), lambda i: (0, 0)),
            pl.BlockSpec((1, 1), lambda i: (0, 0)),
        ],
        out_specs=pl.BlockSpec((BLK, 1), lambda i: (i, 0)),
        out_shape=jax.ShapeDtypeStruct((BATCH, 1), jnp.float32),
    )(u_emb, i_emb, w1u, w1i, b1, w2, b2)


def kernel(user_vector, item_vector, user_table, item_table, W1, b1, W2, b2):
    u_idx = user_vector.reshape(NW * NCHUNK, ICHUNK)
    i_idx = item_vector.reshape(NW * NCHUNK, ICHUNK)
    u_emb, i_emb = _sc_gather(user_table, item_table, u_idx, i_idx)
    return _tc_mlp(u_emb, i_emb, W1[:EMBED], W1[EMBED:],
                   b1.reshape(1, HIDDEN), W2, b2.reshape(1, 1))


# speedup vs baseline: 1.5292x; 1.5292x over previous
import functools

import jax
import jax.numpy as jnp
from jax import lax
from jax.experimental import pallas as pl
from jax.experimental.pallas import tpu as pltpu
from jax.experimental.pallas import tpu_sc as plsc

BATCH = 16384
EMBED = 64
HIDDEN = 256

NC, NS = 2, 16
NW = NC * NS
BPW = BATCH // NW
WROWS = 256
KFLIGHT = 16

_SC_MESH = plsc.VectorSubcoreMesh(
    core_axis_name="c", subcore_axis_name="s", num_cores=NC, num_subcores=NS
)


@functools.partial(
    pl.kernel,
    out_type=(
        jax.ShapeDtypeStruct((BATCH, EMBED), jnp.float32),
        jax.ShapeDtypeStruct((BATCH, EMBED), jnp.float32),
    ),
    mesh=_SC_MESH,
    scratch_types=[
        pltpu.VMEM((BPW,), jnp.int32),
        pltpu.VMEM((BPW,), jnp.int32),
        pltpu.VMEM((WROWS, EMBED), jnp.float32),
        pltpu.VMEM((WROWS, EMBED), jnp.float32),
        pltpu.SemaphoreType.DMA,
        pltpu.SemaphoreType.DMA,
    ],
)
def _sc_gather(u_tab, i_tab, u_idx, i_idx, u_out, i_out,
               uidx_v, iidx_v, urows_v, irows_v, usem, isem):
    wid = lax.axis_index("s") * NC + lax.axis_index("c")
    base = wid * BPW
    pltpu.sync_copy(u_idx.at[pl.ds(base, BPW)], uidx_v)
    pltpu.sync_copy(i_idx.at[pl.ds(base, BPW)], iidx_v)

    for w in range(BPW // WROWS):
        wbase = w * WROWS

        def chunk(c, _, wbase=wbase):
            cbase = c * KFLIGHT
            uvec = uidx_v[pl.ds(wbase + cbase, KFLIGHT)]
            ivec = iidx_v[pl.ds(wbase + cbase, KFLIGHT)]
            for t in range(KFLIGHT):
                j = cbase + t
                pltpu.make_async_copy(
                    u_tab.at[pl.ds(uvec[t], 1)], urows_v.at[pl.ds(j, 1)], usem
                ).start()
                pltpu.make_async_copy(
                    i_tab.at[pl.ds(ivec[t], 1)], irows_v.at[pl.ds(j, 1)], isem
                ).start()
            for t in range(KFLIGHT):
                j = cbase + t
                pltpu.make_async_copy(
                    u_tab.at[pl.ds(uvec[t], 1)], urows_v.at[pl.ds(j, 1)], usem
                ).wait()
                pltpu.make_async_copy(
                    i_tab.at[pl.ds(ivec[t], 1)], irows_v.at[pl.ds(j, 1)], isem
                ).wait()
            return _

        lax.fori_loop(0, WROWS // KFLIGHT, chunk, None)
        pltpu.sync_copy(urows_v, u_out.at[pl.ds(base + wbase, WROWS)])
        pltpu.sync_copy(irows_v, i_out.at[pl.ds(base + wbase, WROWS)])


BLK = 1024


def _mlp_body(u_ref, i_ref, w1u_ref, w1i_ref, b1_ref, w2_ref, b2_ref, o_ref):
    h = (jnp.dot(u_ref[...], w1u_ref[...], preferred_element_type=jnp.float32)
         + jnp.dot(i_ref[...], w1i_ref[...], preferred_element_type=jnp.float32)
         + b1_ref[...])
    h = jnp.maximum(h, 0.0)
    o_ref[...] = (jnp.dot(h, w2_ref[...], preferred_element_type=jnp.float32)
                  + b2_ref[...])


@jax.jit
def _tc_mlp(u_emb, i_emb, w1u, w1i, b1, w2, b2):
    return pl.pallas_call(
        _mlp_body,
        grid=(BATCH // BLK,),
        in_specs=[
            pl.BlockSpec((BLK, EMBED), lambda i: (i, 0)),
            pl.BlockSpec((BLK, EMBED), lambda i: (i, 0)),
            pl.BlockSpec((EMBED, HIDDEN), lambda i: (0, 0)),
            pl.BlockSpec((EMBED, HIDDEN), lambda i: (0, 0)),
            pl.BlockSpec((1, HIDDEN), lambda i: (0, 0)),
            pl.BlockSpec((HIDDEN, 1), lambda i: (0, 0)),
            pl.BlockSpec((1, 1), lambda i: (0, 0)),
        ],
        out_specs=pl.BlockSpec((BLK, 1), lambda i: (i, 0)),
        out_shape=jax.ShapeDtypeStruct((BATCH, 1), jnp.float32),
    )(u_emb, i_emb, w1u, w1i, b1, w2, b2)


def kernel(user_vector, item_vector, user_table, item_table, W1, b1, W2, b2):
    u_emb, i_emb = _sc_gather(user_table, item_table, user_vector, item_vector)
    return _tc_mlp(u_emb, i_emb, W1[:EMBED], W1[EMBED:],
                   b1.reshape(1, HIDDEN), W2, b2.reshape(1, 1))
